# Initial kernel scaffold; baseline (speedup 1.0000x reference)
import jax
import jax.numpy as jnp
from jax.experimental import pallas as pl


def kernel(x, c1w, c1b, c2w, c2b, c3w, c3b, c4w, c4b, c5w, c5b, fc1w, fc1b, fc2w, fc2b, fc3w, fc3b):
    raise NotImplementedError("write your pallas kernel here")


# speedup vs baseline: 2.8618x; 2.8618x over previous
import functools

import jax
import jax.numpy as jnp
from jax.experimental import pallas as pl
from jax.experimental.pallas import tpu as pltpu

BF = jnp.bfloat16
F32 = jnp.float32


def _conv1_pool1_kernel(x_ref, w_ref, b_ref, o_ref, s_ref):
    o_ref[...] = jnp.zeros((1, 968, 64), BF)
    for band in range(3):
        base = 18 * band * 57
        acc = jnp.zeros((1083, 64), F32)
        for t in range(9):
            off = (t // 3) * 57 + (t % 3)
            acc += jnp.dot(x_ref[0, base + off:base + off + 1083, :], w_ref[t],
                           preferred_element_type=F32)
        s_ref[0:1083, :] = jnp.maximum(acc + b_ref[...], 0.0)
        for li in range(9):
            gi = 9 * band + li
            row = None
            for di in range(3):
                for dj in range(3):
                    piece = s_ref[pl.ds((2 * li + di) * 57 + dj, 27, 2), :]
                    row = piece if row is None else jnp.maximum(row, piece)
            o_ref[0, 64 + gi * 31:64 + gi * 31 + 27, :] = row.astype(BF)


def _conv2_pool2_kernel(x_ref, w_ref, b_ref, o_ref, sa_ref, sb_ref):
    o_ref[...] = jnp.zeros((1, 232, 256), BF)
    for band in range(2):
        base = band * 372
        acc = jnp.zeros((465, 256), F32)
        for t in range(25):
            off = base + (t // 5) * 31 + (t % 5)
            acc += jnp.dot(x_ref[0, off:off + 465, :], w_ref[t],
                           preferred_element_type=F32)
        v = jnp.maximum(acc + b_ref[...], 0.0)
        sa_ref[0:465, :] = v[:, 0:128]
        sb_ref[0:465, :] = v[:, 128:256]
        for li in range(7):
            gi = 6 * band + li
            for half, s_ref in enumerate((sa_ref, sb_ref)):
                row = None
                for di in range(3):
                    for dj in range(3):
                        piece = s_ref[pl.ds((2 * li + di) * 31 + dj, 13, 2), :]
                        row = piece if row is None else jnp.maximum(row, piece)
                o_ref[0, 16 + gi * 15:16 + gi * 15 + 13,
                      half * 128:half * 128 + 128] = row.astype(BF)


def _conv3x3_kernel(x_ref, w_ref, b_ref, o_ref, *, cout):
    acc = jnp.zeros((195, cout), F32)
    for t in range(9):
        off = (t // 3) * 15 + (t % 3)
        acc += jnp.dot(x_ref[0, off:off + 195, :], w_ref[t],
                       preferred_element_type=F32)
    v = jnp.maximum(acc + b_ref[...], 0.0).astype(BF)
    col = jax.lax.broadcasted_iota(jnp.int32, (195, cout), 0) % 15
    v = jnp.where(col < 13, v, jnp.zeros_like(v))
    o_ref[0, 0:16, :] = jnp.zeros((16, cout), BF)
    o_ref[0, 16:211, :] = v
    o_ref[0, 211:232, :] = jnp.zeros((21, cout), BF)


def _conv5_pool3_kernel(x_ref, w_ref, b_ref, o_ref, sa_ref, sb_ref):
    acc = jnp.zeros((195, 256), F32)
    for t in range(9):
        off = (t // 3) * 15 + (t % 3)
        acc += jnp.dot(x_ref[0, off:off + 195, :], w_ref[t],
                       preferred_element_type=F32)
    v = jnp.maximum(acc + b_ref[...], 0.0)
    sa_ref[0:195, :] = v[:, 0:128]
    sb_ref[0:195, :] = v[:, 128:256]
    for i in range(6):
        for half, s_ref in enumerate((sa_ref, sb_ref)):
            row = None
            for di in range(3):
                for dj in range(3):
                    piece = s_ref[pl.ds((2 * i + di) * 15 + dj, 6, 2), :]
                    row = piece if row is None else jnp.maximum(row, piece)
            o_ref[0, 6 * i:6 * i + 6, half * 128:half * 128 + 128] = row.astype(BF)


def _fc_kernel(a_ref, w_ref, b_ref, o_ref, *, relu):
    r = jnp.dot(a_ref[...], w_ref[...], preferred_element_type=F32) + b_ref[...]
    if relu:
        r = jnp.maximum(r, 0.0)
    o_ref[...] = r.astype(o_ref.dtype)


def _fc(a, w, b, *, relu, tn, out_dtype):
    M, K = a.shape
    N = w.shape[1]
    return pl.pallas_call(
        functools.partial(_fc_kernel, relu=relu),
        out_shape=jax.ShapeDtypeStruct((M, N), out_dtype),
        grid=(N // tn,),
        in_specs=[
            pl.BlockSpec((M, K), lambda j: (0, 0)),
            pl.BlockSpec((K, tn), lambda j: (0, j)),
            pl.BlockSpec((1, tn), lambda j: (0, j)),
        ],
        out_specs=pl.BlockSpec((M, tn), lambda j: (0, j)),
        compiler_params=pltpu.CompilerParams(
            dimension_semantics=("parallel",),
            vmem_limit_bytes=48 * 1024 * 1024),
    )(a, w, b.reshape(1, N).astype(F32))


def kernel(x, c1w, c1b, c2w, c2b, c3w, c3b, c4w, c4b, c5w, c5b,
           fc1w, fc1b, fc2w, fc2b, fc3w, fc3b):
    n = x.shape[0]

    xp = jnp.pad(x, ((0, 0), (0, 0), (2, 2), (2, 2)))
    xs = xp.reshape(n, 3, 57, 4, 57, 4).transpose(0, 2, 4, 3, 5, 1)
    xs = jnp.pad(xs.reshape(n, 3249, 48), ((0, 0), (0, 7), (0, 0))).astype(BF)

    w1 = jnp.pad(c1w, ((0, 1), (0, 1), (0, 0), (0, 0)))
    w1 = w1.reshape(3, 4, 3, 4, 3, 64).transpose(0, 2, 1, 3, 4, 5)
    w1 = w1.reshape(9, 48, 64)

    w2 = jnp.pad(c2w, ((0, 0), (0, 0), (0, 0), (0, 64))).reshape(25, 64, 256)
    b2 = jnp.pad(c2b, (0, 64))
    w3 = jnp.pad(c3w, ((0, 0), (0, 0), (0, 64), (0, 0))).reshape(9, 256, 384)
    w4 = c4w.reshape(9, 384, 256)
    w5 = c5w.reshape(9, 256, 256)

    par = pltpu.CompilerParams(dimension_semantics=("parallel",))

    h = pl.pallas_call(
        _conv1_pool1_kernel,
        out_shape=jax.ShapeDtypeStruct((n, 968, 64), BF),
        grid=(n,),
        in_specs=[
            pl.BlockSpec((1, 3256, 48), lambda i: (i, 0, 0)),
            pl.BlockSpec((9, 48, 64), lambda i: (0, 0, 0)),
            pl.BlockSpec((1, 64), lambda i: (0, 0)),
        ],
        out_specs=pl.BlockSpec((1, 968, 64), lambda i: (i, 0, 0)),
        scratch_shapes=[pltpu.VMEM((1088, 64), F32)],
        compiler_params=par,
    )(xs, w1, c1b.reshape(1, 64).astype(F32))

    h = pl.pallas_call(
        _conv2_pool2_kernel,
        out_shape=jax.ShapeDtypeStruct((n, 232, 256), BF),
        grid=(n,),
        in_specs=[
            pl.BlockSpec((1, 968, 64), lambda i: (i, 0, 0)),
            pl.BlockSpec((25, 64, 256), lambda i: (0, 0, 0)),
            pl.BlockSpec((1, 256), lambda i: (0, 0)),
        ],
        out_specs=pl.BlockSpec((1, 232, 256), lambda i: (i, 0, 0)),
        scratch_shapes=[pltpu.VMEM((472, 128), F32), pltpu.VMEM((472, 128), F32)],
        compiler_params=par,
    )(h, w2, b2.reshape(1, 256).astype(F32))

    h = pl.pallas_call(
        functools.partial(_conv3x3_kernel, cout=384),
        out_shape=jax.ShapeDtypeStruct((n, 232, 384), BF),
        grid=(n,),
        in_specs=[
            pl.BlockSpec((1, 232, 256), lambda i: (i, 0, 0)),
            pl.BlockSpec((9, 256, 384), lambda i: (0, 0, 0)),
            pl.BlockSpec((1, 384), lambda i: (0, 0)),
        ],
        out_specs=pl.BlockSpec((1, 232, 384), lambda i: (i, 0, 0)),
        compiler_params=par,
    )(h, w3, c3b.reshape(1, 384).astype(F32))

    h = pl.pallas_call(
        functools.partial(_conv3x3_kernel, cout=256),
        out_shape=jax.ShapeDtypeStruct((n, 232, 256), BF),
        grid=(n,),
        in_specs=[
            pl.BlockSpec((1, 232, 384), lambda i: (i, 0, 0)),
            pl.BlockSpec((9, 384, 256), lambda i: (0, 0, 0)),
            pl.BlockSpec((1, 256), lambda i: (0, 0)),
        ],
        out_specs=pl.BlockSpec((1, 232, 256), lambda i: (i, 0, 0)),
        compiler_params=par,
    )(h, w4, c4b.reshape(1, 256).astype(F32))

    h = pl.pallas_call(
        _conv5_pool3_kernel,
        out_shape=jax.ShapeDtypeStruct((n, 36, 256), BF),
        grid=(n,),
        in_specs=[
            pl.BlockSpec((1, 232, 256), lambda i: (i, 0, 0)),
            pl.BlockSpec((9, 256, 256), lambda i: (0, 0, 0)),
            pl.BlockSpec((1, 256), lambda i: (0, 0)),
        ],
        out_specs=pl.BlockSpec((1, 36, 256), lambda i: (i, 0, 0)),
        scratch_shapes=[pltpu.VMEM((200, 128), F32), pltpu.VMEM((200, 128), F32)],
        compiler_params=par,
    )(h, w5, c5b.reshape(1, 256).astype(F32))

    flat = h.transpose(0, 2, 1).reshape(n, 9216)

    h1 = _fc(flat, fc1w, fc1b, relu=True, tn=512, out_dtype=BF)
    h2 = _fc(h1, fc2w, fc2b, relu=True, tn=512, out_dtype=BF)
    w3f = jnp.pad(fc3w, ((0, 0), (0, 114)))
    b3f = jnp.pad(fc3b, (0, 114))
    h3 = _fc(h2, w3f, b3f, relu=False, tn=128, out_dtype=F32)
    return h3[:, :14]


# speedup vs baseline: 3.3470x; 1.1696x over previous
import functools

import jax
import jax.numpy as jnp
from jax.experimental import pallas as pl
from jax.experimental.pallas import tpu as pltpu

BF = jnp.bfloat16
F32 = jnp.float32


def _conv1_pool1_kernel(x_ref, w_ref, b_ref, o_ref, s_ref):
    o_ref[...] = jnp.zeros((1, 968, 64), BF)
    for band in range(3):
        base = 18 * band * 57
        acc = jnp.zeros((1083, 64), F32)
        for t in range(9):
            off = (t // 3) * 57 + (t % 3)
            acc += jnp.dot(x_ref[0, base + off:base + off + 1083, :], w_ref[t],
                           preferred_element_type=F32)
        s_ref[0:1083, :] = jnp.maximum(acc + b_ref[...], 0.0)
        for li in range(9):
            gi = 9 * band + li
            row = None
            for di in range(3):
                for dj in range(3):
                    piece = s_ref[pl.ds((2 * li + di) * 57 + dj, 27, 2), :]
                    row = piece if row is None else jnp.maximum(row, piece)
            o_ref[0, 64 + gi * 31:64 + gi * 31 + 27, :] = row.astype(BF)


def _conv2_pool2_kernel(x_ref, w_ref, b_ref, o_ref, sa_ref, sb_ref):
    o_ref[...] = jnp.zeros((1, 232, 256), BF)
    for band in range(2):
        base = band * 372
        acc = jnp.zeros((465, 256), F32)
        for t in range(25):
            off = base + (t // 5) * 31 + (t % 5)
            acc += jnp.dot(x_ref[0, off:off + 465, :], w_ref[t],
                           preferred_element_type=F32)
        v = jnp.maximum(acc + b_ref[...], 0.0)
        sa_ref[0:465, :] = v[:, 0:128]
        sb_ref[0:465, :] = v[:, 128:256]
        for li in range(7):
            gi = 6 * band + li
            for half, s_ref in enumerate((sa_ref, sb_ref)):
                row = None
                for di in range(3):
                    for dj in range(3):
                        piece = s_ref[pl.ds((2 * li + di) * 31 + dj, 13, 2), :]
                        row = piece if row is None else jnp.maximum(row, piece)
                o_ref[0, 16 + gi * 15:16 + gi * 15 + 13,
                      half * 128:half * 128 + 128] = row.astype(BF)


def _conv3x3_kernel(x_ref, w_ref, b_ref, o_ref, *, cout):
    acc = jnp.zeros((195, cout), F32)
    for t in range(9):
        off = (t // 3) * 15 + (t % 3)
        acc += jnp.dot(x_ref[0, off:off + 195, :], w_ref[t],
                       preferred_element_type=F32)
    v = jnp.maximum(acc + b_ref[...], 0.0).astype(BF)
    col = jax.lax.broadcasted_iota(jnp.int32, (195, cout), 0) % 15
    v = jnp.where(col < 13, v, jnp.zeros_like(v))
    o_ref[0, 0:16, :] = jnp.zeros((16, cout), BF)
    o_ref[0, 16:211, :] = v
    o_ref[0, 211:232, :] = jnp.zeros((21, cout), BF)


def _conv5_pool3_kernel(x_ref, w_ref, b_ref, o_ref, sa_ref, sb_ref):
    acc = jnp.zeros((195, 256), F32)
    for t in range(9):
        off = (t // 3) * 15 + (t % 3)
        acc += jnp.dot(x_ref[0, off:off + 195, :], w_ref[t],
                       preferred_element_type=F32)
    v = jnp.maximum(acc + b_ref[...], 0.0)
    sa_ref[0:195, :] = v[:, 0:128]
    sb_ref[0:195, :] = v[:, 128:256]
    for i in range(6):
        for half, s_ref in enumerate((sa_ref, sb_ref)):
            row = None
            for di in range(3):
                for dj in range(3):
                    piece = s_ref[pl.ds((2 * i + di) * 15 + dj, 6, 2), :]
                    row = piece if row is None else jnp.maximum(row, piece)
            o_ref[0, 6 * i:6 * i + 6, half * 128:half * 128 + 128] = row.astype(BF)


def _fc_kernel(a_ref, w_ref, b_ref, o_ref, *, relu):
    r = jnp.dot(a_ref[...], w_ref[...], preferred_element_type=F32) + b_ref[...]
    if relu:
        r = jnp.maximum(r, 0.0)
    o_ref[...] = r.astype(o_ref.dtype)


def _fc(a, w, b, *, relu, tn, out_dtype):
    M, K = a.shape
    N = w.shape[1]
    return pl.pallas_call(
        functools.partial(_fc_kernel, relu=relu),
        out_shape=jax.ShapeDtypeStruct((M, N), out_dtype),
        grid=(N // tn,),
        in_specs=[
            pl.BlockSpec((M, K), lambda j: (0, 0)),
            pl.BlockSpec((K, tn), lambda j: (0, j)),
            pl.BlockSpec((1, tn), lambda j: (0, j)),
        ],
        out_specs=pl.BlockSpec((M, tn), lambda j: (0, j)),
        compiler_params=pltpu.CompilerParams(
            dimension_semantics=("parallel",),
            vmem_limit_bytes=48 * 1024 * 1024),
    )(a, w, b.reshape(1, N).astype(F32))


def kernel(x, c1w, c1b, c2w, c2b, c3w, c3b, c4w, c4b, c5w, c5b,
           fc1w, fc1b, fc2w, fc2b, fc3w, fc3b):
    n = x.shape[0]

    xa = jnp.pad(x.transpose(0, 2, 3, 1).astype(BF),
                 ((0, 0), (2, 2), (2, 2), (0, 0)))
    xa = xa.reshape(n, 228, 57, 12).transpose(0, 2, 1, 3)
    xa = xa.reshape(n, 57, 57, 4, 12).transpose(0, 2, 1, 3, 4)
    xs = jnp.pad(xa.reshape(n, 3249, 48), ((0, 0), (0, 7), (0, 0)))

    w1 = jnp.pad(c1w, ((0, 1), (0, 1), (0, 0), (0, 0)))
    w1 = w1.reshape(3, 4, 3, 4, 3, 64).transpose(0, 2, 1, 3, 4, 5)
    w1 = w1.reshape(9, 48, 64)

    w2 = jnp.pad(c2w, ((0, 0), (0, 0), (0, 0), (0, 64))).reshape(25, 64, 256)
    b2 = jnp.pad(c2b, (0, 64))
    w3 = jnp.pad(c3w, ((0, 0), (0, 0), (0, 64), (0, 0))).reshape(9, 256, 384)
    w4 = c4w.reshape(9, 384, 256)
    w5 = c5w.reshape(9, 256, 256)

    par = pltpu.CompilerParams(dimension_semantics=("parallel",))

    h = pl.pallas_call(
        _conv1_pool1_kernel,
        out_shape=jax.ShapeDtypeStruct((n, 968, 64), BF),
        grid=(n,),
        in_specs=[
            pl.BlockSpec((1, 3256, 48), lambda i: (i, 0, 0)),
            pl.BlockSpec((9, 48, 64), lambda i: (0, 0, 0)),
            pl.BlockSpec((1, 64), lambda i: (0, 0)),
        ],
        out_specs=pl.BlockSpec((1, 968, 64), lambda i: (i, 0, 0)),
        scratch_shapes=[pltpu.VMEM((1088, 64), F32)],
        compiler_params=par,
    )(xs, w1, c1b.reshape(1, 64).astype(F32))

    h = pl.pallas_call(
        _conv2_pool2_kernel,
        out_shape=jax.ShapeDtypeStruct((n, 232, 256), BF),
        grid=(n,),
        in_specs=[
            pl.BlockSpec((1, 968, 64), lambda i: (i, 0, 0)),
            pl.BlockSpec((25, 64, 256), lambda i: (0, 0, 0)),
            pl.BlockSpec((1, 256), lambda i: (0, 0)),
        ],
        out_specs=pl.BlockSpec((1, 232, 256), lambda i: (i, 0, 0)),
        scratch_shapes=[pltpu.VMEM((472, 128), F32), pltpu.VMEM((472, 128), F32)],
        compiler_params=par,
    )(h, w2, b2.reshape(1, 256).astype(F32))

    h = pl.pallas_call(
        functools.partial(_conv3x3_kernel, cout=384),
        out_shape=jax.ShapeDtypeStruct((n, 232, 384), BF),
        grid=(n,),
        in_specs=[
            pl.BlockSpec((1, 232, 256), lambda i: (i, 0, 0)),
            pl.BlockSpec((9, 256, 384), lambda i: (0, 0, 0)),
            pl.BlockSpec((1, 384), lambda i: (0, 0)),
        ],
        out_specs=pl.BlockSpec((1, 232, 384), lambda i: (i, 0, 0)),
        compiler_params=par,
    )(h, w3, c3b.reshape(1, 384).astype(F32))

    h = pl.pallas_call(
        functools.partial(_conv3x3_kernel, cout=256),
        out_shape=jax.ShapeDtypeStruct((n, 232, 256), BF),
        grid=(n,),
        in_specs=[
            pl.BlockSpec((1, 232, 384), lambda i: (i, 0, 0)),
            pl.BlockSpec((9, 384, 256), lambda i: (0, 0, 0)),
            pl.BlockSpec((1, 256), lambda i: (0, 0)),
        ],
        out_specs=pl.BlockSpec((1, 232, 256), lambda i: (i, 0, 0)),
        compiler_params=par,
    )(h, w4, c4b.reshape(1, 256).astype(F32))

    h = pl.pallas_call(
        _conv5_pool3_kernel,
        out_shape=jax.ShapeDtypeStruct((n, 36, 256), BF),
        grid=(n,),
        in_specs=[
            pl.BlockSpec((1, 232, 256), lambda i: (i, 0, 0)),
            pl.BlockSpec((9, 256, 256), lambda i: (0, 0, 0)),
            pl.BlockSpec((1, 256), lambda i: (0, 0)),
        ],
        out_specs=pl.BlockSpec((1, 36, 256), lambda i: (i, 0, 0)),
        scratch_shapes=[pltpu.VMEM((200, 128), F32), pltpu.VMEM((200, 128), F32)],
        compiler_params=par,
    )(h, w5, c5b.reshape(1, 256).astype(F32))

    flat = h.transpose(0, 2, 1).reshape(n, 9216)

    h1 = _fc(flat, fc1w, fc1b, relu=True, tn=512, out_dtype=BF)
    h2 = _fc(h1, fc2w, fc2b, relu=True, tn=512, out_dtype=BF)
    w3f = jnp.pad(fc3w, ((0, 0), (0, 114)))
    b3f = jnp.pad(fc3b, (0, 114))
    h3 = _fc(h2, w3f, b3f, relu=False, tn=128, out_dtype=F32)
    return h3[:, :14]


# speedup vs baseline: 5.6054x; 1.6747x over previous
import functools

import jax
import jax.numpy as jnp
from jax.experimental import pallas as pl
from jax.experimental.pallas import tpu as pltpu

BF = jnp.bfloat16
F32 = jnp.float32


def _conv1_pool1_kernel(x_ref, w_ref, b_ref, o_ref, s_ref):
    o_ref[...] = jnp.zeros((1, 968, 64), BF)
    for band in range(3):
        base = 18 * band * 57
        acc = jnp.zeros((1083, 64), F32)
        for t in range(9):
            off = (t // 3) * 57 + (t % 3)
            acc += jnp.dot(x_ref[0, base + off:base + off + 1083, :], w_ref[t],
                           preferred_element_type=F32)
        s_ref[0:1083, :] = jnp.maximum(acc + b_ref[...], 0.0)
        for li in range(9):
            gi = 9 * band + li
            row = None
            for di in range(3):
                for dj in range(3):
                    piece = s_ref[pl.ds((2 * li + di) * 57 + dj, 27, 2), :]
                    row = piece if row is None else jnp.maximum(row, piece)
            o_ref[0, 64 + gi * 31:64 + gi * 31 + 27, :] = row.astype(BF)


def _conv2_pool2_kernel(x_ref, w_ref, b_ref, o_ref, sa_ref, sb_ref):
    o_ref[...] = jnp.zeros((1, 232, 256), BF)
    for band in range(2):
        base = band * 372
        acc = jnp.zeros((465, 256), F32)
        for t in range(25):
            off = base + (t // 5) * 31 + (t % 5)
            acc += jnp.dot(x_ref[0, off:off + 465, :], w_ref[t],
                           preferred_element_type=F32)
        v = jnp.maximum(acc + b_ref[...], 0.0)
        sa_ref[0:465, :] = v[:, 0:128]
        sb_ref[0:465, :] = v[:, 128:256]
        for li in range(7):
            gi = 6 * band + li
            for half, s_ref in enumerate((sa_ref, sb_ref)):
                row = None
                for di in range(3):
                    for dj in range(3):
                        piece = s_ref[pl.ds((2 * li + di) * 31 + dj, 13, 2), :]
                        row = piece if row is None else jnp.maximum(row, piece)
                o_ref[0, 16 + gi * 15:16 + gi * 15 + 13,
                      half * 128:half * 128 + 128] = row.astype(BF)


def _conv3x3_kernel(x_ref, w_ref, b_ref, o_ref, *, cout):
    acc = jnp.zeros((195, cout), F32)
    for t in range(9):
        off = (t // 3) * 15 + (t % 3)
        acc += jnp.dot(x_ref[0, off:off + 195, :], w_ref[t],
                       preferred_element_type=F32)
    v = jnp.maximum(acc + b_ref[...], 0.0).astype(BF)
    col = jax.lax.broadcasted_iota(jnp.int32, (195, cout), 0) % 15
    v = jnp.where(col < 13, v, jnp.zeros_like(v))
    o_ref[0, 0:16, :] = jnp.zeros((16, cout), BF)
    o_ref[0, 16:211, :] = v
    o_ref[0, 211:232, :] = jnp.zeros((21, cout), BF)


def _conv5_pool3_kernel(x_ref, w_ref, b_ref, o_ref, sa_ref, sb_ref):
    acc = jnp.zeros((195, 256), F32)
    for t in range(9):
        off = (t // 3) * 15 + (t % 3)
        acc += jnp.dot(x_ref[0, off:off + 195, :], w_ref[t],
                       preferred_element_type=F32)
    v = jnp.maximum(acc + b_ref[...], 0.0)
    sa_ref[0:195, :] = v[:, 0:128]
    sb_ref[0:195, :] = v[:, 128:256]
    for i in range(6):
        for half, s_ref in enumerate((sa_ref, sb_ref)):
            row = None
            for di in range(3):
                for dj in range(3):
                    piece = s_ref[pl.ds((2 * i + di) * 15 + dj, 6, 2), :]
                    row = piece if row is None else jnp.maximum(row, piece)
            o_ref[0, 6 * i:6 * i + 6, half * 128:half * 128 + 128] = row.astype(BF)


def _fc_kernel(a_ref, w_ref, b_ref, o_ref, *, relu):
    r = jnp.dot(a_ref[...], w_ref[...], preferred_element_type=F32) + b_ref[...]
    if relu:
        r = jnp.maximum(r, 0.0)
    o_ref[...] = r.astype(o_ref.dtype)


def _fc(a, w, b, *, relu, tn, out_dtype):
    M, K = a.shape
    N = w.shape[1]
    return pl.pallas_call(
        functools.partial(_fc_kernel, relu=relu),
        out_shape=jax.ShapeDtypeStruct((M, N), out_dtype),
        grid=(N // tn,),
        in_specs=[
            pl.BlockSpec((M, K), lambda j: (0, 0)),
            pl.BlockSpec((K, tn), lambda j: (0, j)),
            pl.BlockSpec((1, tn), lambda j: (0, j)),
        ],
        out_specs=pl.BlockSpec((M, tn), lambda j: (0, j)),
        compiler_params=pltpu.CompilerParams(
            dimension_semantics=("parallel",),
            vmem_limit_bytes=48 * 1024 * 1024),
    )(a, w, b.reshape(1, N).astype(F32))


def kernel(x, c1w, c1b, c2w, c2b, c3w, c3b, c4w, c4b, c5w, c5b,
           fc1w, fc1b, fc2w, fc2b, fc3w, fc3b):
    n = x.shape[0]

    xa = jnp.pad(x.transpose(0, 2, 3, 1).astype(BF),
                 ((0, 0), (2, 2), (2, 2), (0, 0)))
    xa = xa.reshape(n, 57, 4, 57, 12)
    i4 = jnp.eye(4, dtype=BF)
    i12 = jnp.eye(12, dtype=BF)
    xa = jnp.einsum('nhuws,uv->nhwsv', xa, i4)
    xa = jnp.einsum('nhwsv,st->nhwvt', xa, i12)
    xs = jnp.pad(xa.reshape(n, 3249, 48), ((0, 0), (0, 7), (0, 0)))

    w1 = jnp.pad(c1w, ((0, 1), (0, 1), (0, 0), (0, 0)))
    w1 = w1.reshape(3, 4, 3, 4, 3, 64).transpose(0, 2, 1, 3, 4, 5)
    w1 = w1.reshape(9, 48, 64)

    w2 = jnp.pad(c2w, ((0, 0), (0, 0), (0, 0), (0, 64))).reshape(25, 64, 256)
    b2 = jnp.pad(c2b, (0, 64))
    w3 = jnp.pad(c3w, ((0, 0), (0, 0), (0, 64), (0, 0))).reshape(9, 256, 384)
    w4 = c4w.reshape(9, 384, 256)
    w5 = c5w.reshape(9, 256, 256)

    par = pltpu.CompilerParams(dimension_semantics=("parallel",))

    h = pl.pallas_call(
        _conv1_pool1_kernel,
        out_shape=jax.ShapeDtypeStruct((n, 968, 64), BF),
        grid=(n,),
        in_specs=[
            pl.BlockSpec((1, 3256, 48), lambda i: (i, 0, 0)),
            pl.BlockSpec((9, 48, 64), lambda i: (0, 0, 0)),
            pl.BlockSpec((1, 64), lambda i: (0, 0)),
        ],
        out_specs=pl.BlockSpec((1, 968, 64), lambda i: (i, 0, 0)),
        scratch_shapes=[pltpu.VMEM((1088, 64), F32)],
        compiler_params=par,
    )(xs, w1, c1b.reshape(1, 64).astype(F32))

    h = pl.pallas_call(
        _conv2_pool2_kernel,
        out_shape=jax.ShapeDtypeStruct((n, 232, 256), BF),
        grid=(n,),
        in_specs=[
            pl.BlockSpec((1, 968, 64), lambda i: (i, 0, 0)),
            pl.BlockSpec((25, 64, 256), lambda i: (0, 0, 0)),
            pl.BlockSpec((1, 256), lambda i: (0, 0)),
        ],
        out_specs=pl.BlockSpec((1, 232, 256), lambda i: (i, 0, 0)),
        scratch_shapes=[pltpu.VMEM((472, 128), F32), pltpu.VMEM((472, 128), F32)],
        compiler_params=par,
    )(h, w2, b2.reshape(1, 256).astype(F32))

    h = pl.pallas_call(
        functools.partial(_conv3x3_kernel, cout=384),
        out_shape=jax.ShapeDtypeStruct((n, 232, 384), BF),
        grid=(n,),
        in_specs=[
            pl.BlockSpec((1, 232, 256), lambda i: (i, 0, 0)),
            pl.BlockSpec((9, 256, 384), lambda i: (0, 0, 0)),
            pl.BlockSpec((1, 384), lambda i: (0, 0)),
        ],
        out_specs=pl.BlockSpec((1, 232, 384), lambda i: (i, 0, 0)),
        compiler_params=par,
    )(h, w3, c3b.reshape(1, 384).astype(F32))

    h = pl.pallas_call(
        functools.partial(_conv3x3_kernel, cout=256),
        out_shape=jax.ShapeDtypeStruct((n, 232, 256), BF),
        grid=(n,),
        in_specs=[
            pl.BlockSpec((1, 232, 384), lambda i: (i, 0, 0)),
            pl.BlockSpec((9, 384, 256), lambda i: (0, 0, 0)),
            pl.BlockSpec((1, 256), lambda i: (0, 0)),
        ],
        out_specs=pl.BlockSpec((1, 232, 256), lambda i: (i, 0, 0)),
        compiler_params=par,
    )(h, w4, c4b.reshape(1, 256).astype(F32))

    h = pl.pallas_call(
        _conv5_pool3_kernel,
        out_shape=jax.ShapeDtypeStruct((n, 36, 256), BF),
        grid=(n,),
        in_specs=[
            pl.BlockSpec((1, 232, 256), lambda i: (i, 0, 0)),
            pl.BlockSpec((9, 256, 256), lambda i: (0, 0, 0)),
            pl.BlockSpec((1, 256), lambda i: (0, 0)),
        ],
        out_specs=pl.BlockSpec((1, 36, 256), lambda i: (i, 0, 0)),
        scratch_shapes=[pltpu.VMEM((200, 128), F32), pltpu.VMEM((200, 128), F32)],
        compiler_params=par,
    )(h, w5, c5b.reshape(1, 256).astype(F32))

    flat = h.transpose(0, 2, 1).reshape(n, 9216)

    h1 = _fc(flat, fc1w, fc1b, relu=True, tn=512, out_dtype=BF)
    h2 = _fc(h1, fc2w, fc2b, relu=True, tn=512, out_dtype=BF)
    w3f = jnp.pad(fc3w, ((0, 0), (0, 114)))
    b3f = jnp.pad(fc3b, (0, 114))
    h3 = _fc(h2, w3f, b3f, relu=False, tn=128, out_dtype=F32)
    return h3[:, :14]


# speedup vs baseline: 13.6223x; 2.4302x over previous
import functools

import jax
import jax.numpy as jnp
from jax.experimental import pallas as pl
from jax.experimental.pallas import tpu as pltpu

BF = jnp.bfloat16
F32 = jnp.float32


def _conv1_pool1_kernel(x_ref, w_ref, b_ref, o_ref, s_ref):
    o_ref[...] = jnp.zeros((1, 968, 64), BF)
    for band in range(3):
        base = 18 * band * 57
        acc = jnp.zeros((1083, 64), F32)
        for t in range(9):
            off = (t // 3) * 57 + (t % 3)
            acc += jnp.dot(x_ref[0, base + off:base + off + 1083, :], w_ref[t],
                           preferred_element_type=F32)
        s_ref[0:1083, :] = jnp.maximum(acc + b_ref[...], 0.0)
        for li in range(9):
            gi = 9 * band + li
            row = None
            for di in range(3):
                for dj in range(3):
                    piece = s_ref[pl.ds((2 * li + di) * 57 + dj, 27, 2), :]
                    row = piece if row is None else jnp.maximum(row, piece)
            o_ref[0, 64 + gi * 31:64 + gi * 31 + 27, :] = row.astype(BF)


def _conv2_pool2_kernel(x_ref, w_ref, b_ref, o_ref, sa_ref, sb_ref):
    o_ref[...] = jnp.zeros((1, 232, 256), BF)
    for band in range(2):
        base = band * 372
        acc = jnp.zeros((465, 256), F32)
        for t in range(25):
            off = base + (t // 5) * 31 + (t % 5)
            acc += jnp.dot(x_ref[0, off:off + 465, :], w_ref[t],
                           preferred_element_type=F32)
        v = jnp.maximum(acc + b_ref[...], 0.0)
        sa_ref[0:465, :] = v[:, 0:128]
        sb_ref[0:465, :] = v[:, 128:256]
        for li in range(7):
            gi = 6 * band + li
            for half, s_ref in enumerate((sa_ref, sb_ref)):
                row = None
                for di in range(3):
                    for dj in range(3):
                        piece = s_ref[pl.ds((2 * li + di) * 31 + dj, 13, 2), :]
                        row = piece if row is None else jnp.maximum(row, piece)
                o_ref[0, 16 + gi * 15:16 + gi * 15 + 13,
                      half * 128:half * 128 + 128] = row.astype(BF)


def _conv3x3_kernel(x_ref, w_ref, b_ref, o_ref, *, cout):
    acc = jnp.zeros((195, cout), F32)
    for t in range(9):
        off = (t // 3) * 15 + (t % 3)
        acc += jnp.dot(x_ref[0, off:off + 195, :], w_ref[t],
                       preferred_element_type=F32)
    v = jnp.maximum(acc + b_ref[...], 0.0).astype(BF)
    col = jax.lax.broadcasted_iota(jnp.int32, (195, cout), 0) % 15
    v = jnp.where(col < 13, v, jnp.zeros_like(v))
    o_ref[0, 0:16, :] = jnp.zeros((16, cout), BF)
    o_ref[0, 16:211, :] = v
    o_ref[0, 211:232, :] = jnp.zeros((21, cout), BF)


def _conv5_pool3_kernel(x_ref, w_ref, b_ref, o_ref, sa_ref, sb_ref):
    acc = jnp.zeros((195, 256), F32)
    for t in range(9):
        off = (t // 3) * 15 + (t % 3)
        acc += jnp.dot(x_ref[0, off:off + 195, :], w_ref[t],
                       preferred_element_type=F32)
    v = jnp.maximum(acc + b_ref[...], 0.0)
    sa_ref[0:195, :] = v[:, 0:128]
    sb_ref[0:195, :] = v[:, 128:256]
    for i in range(6):
        for half, s_ref in enumerate((sa_ref, sb_ref)):
            row = None
            for di in range(3):
                for dj in range(3):
                    piece = s_ref[pl.ds((2 * i + di) * 15 + dj, 6, 2), :]
                    row = piece if row is None else jnp.maximum(row, piece)
            o_ref[0, 6 * i:6 * i + 6, half * 128:half * 128 + 128] = row.astype(BF)


def _fc_kernel(a_ref, w_ref, b_ref, o_ref, *, relu):
    r = jnp.dot(a_ref[...], w_ref[...], preferred_element_type=F32) + b_ref[...]
    if relu:
        r = jnp.maximum(r, 0.0)
    o_ref[...] = r.astype(o_ref.dtype)


def _fc(a, w, b, *, relu, tn, out_dtype):
    M, K = a.shape
    N = w.shape[1]
    return pl.pallas_call(
        functools.partial(_fc_kernel, relu=relu),
        out_shape=jax.ShapeDtypeStruct((M, N), out_dtype),
        grid=(N // tn,),
        in_specs=[
            pl.BlockSpec((M, K), lambda j: (0, 0)),
            pl.BlockSpec((K, tn), lambda j: (0, j)),
            pl.BlockSpec((1, tn), lambda j: (0, j)),
        ],
        out_specs=pl.BlockSpec((M, tn), lambda j: (0, j)),
        compiler_params=pltpu.CompilerParams(
            dimension_semantics=("parallel",),
            vmem_limit_bytes=48 * 1024 * 1024),
    )(a, w, b.reshape(1, N).astype(F32))


def kernel(x, c1w, c1b, c2w, c2b, c3w, c3b, c4w, c4b, c5w, c5b,
           fc1w, fc1b, fc2w, fc2b, fc3w, fc3b):
    n = x.shape[0]

    xa = jnp.pad(x.transpose(0, 2, 3, 1).astype(BF),
                 ((0, 0), (2, 2), (2, 2), (0, 0)))
    phases = [xa[:, uh::4, :, :].reshape(n, 3249, 12) for uh in range(4)]
    xs = jnp.pad(jnp.concatenate(phases, axis=-1), ((0, 0), (0, 7), (0, 0)))

    w1 = jnp.pad(c1w, ((0, 1), (0, 1), (0, 0), (0, 0)))
    w1 = w1.reshape(3, 4, 3, 4, 3, 64).transpose(0, 2, 1, 3, 4, 5)
    w1 = w1.reshape(9, 48, 64)

    w2 = jnp.pad(c2w, ((0, 0), (0, 0), (0, 0), (0, 64))).reshape(25, 64, 256)
    b2 = jnp.pad(c2b, (0, 64))
    w3 = jnp.pad(c3w, ((0, 0), (0, 0), (0, 64), (0, 0))).reshape(9, 256, 384)
    w4 = c4w.reshape(9, 384, 256)
    w5 = c5w.reshape(9, 256, 256)

    par = pltpu.CompilerParams(dimension_semantics=("parallel",))

    h = pl.pallas_call(
        _conv1_pool1_kernel,
        out_shape=jax.ShapeDtypeStruct((n, 968, 64), BF),
        grid=(n,),
        in_specs=[
            pl.BlockSpec((1, 3256, 48), lambda i: (i, 0, 0)),
            pl.BlockSpec((9, 48, 64), lambda i: (0, 0, 0)),
            pl.BlockSpec((1, 64), lambda i: (0, 0)),
        ],
        out_specs=pl.BlockSpec((1, 968, 64), lambda i: (i, 0, 0)),
        scratch_shapes=[pltpu.VMEM((1088, 64), F32)],
        compiler_params=par,
    )(xs, w1, c1b.reshape(1, 64).astype(F32))

    h = pl.pallas_call(
        _conv2_pool2_kernel,
        out_shape=jax.ShapeDtypeStruct((n, 232, 256), BF),
        grid=(n,),
        in_specs=[
            pl.BlockSpec((1, 968, 64), lambda i: (i, 0, 0)),
            pl.BlockSpec((25, 64, 256), lambda i: (0, 0, 0)),
            pl.BlockSpec((1, 256), lambda i: (0, 0)),
        ],
        out_specs=pl.BlockSpec((1, 232, 256), lambda i: (i, 0, 0)),
        scratch_shapes=[pltpu.VMEM((472, 128), F32), pltpu.VMEM((472, 128), F32)],
        compiler_params=par,
    )(h, w2, b2.reshape(1, 256).astype(F32))

    h = pl.pallas_call(
        functools.partial(_conv3x3_kernel, cout=384),
        out_shape=jax.ShapeDtypeStruct((n, 232, 384), BF),
        grid=(n,),
        in_specs=[
            pl.BlockSpec((1, 232, 256), lambda i: (i, 0, 0)),
            pl.BlockSpec((9, 256, 384), lambda i: (0, 0, 0)),
            pl.BlockSpec((1, 384), lambda i: (0, 0)),
        ],
        out_specs=pl.BlockSpec((1, 232, 384), lambda i: (i, 0, 0)),
        compiler_params=par,
    )(h, w3, c3b.reshape(1, 384).astype(F32))

    h = pl.pallas_call(
        functools.partial(_conv3x3_kernel, cout=256),
        out_shape=jax.ShapeDtypeStruct((n, 232, 256), BF),
        grid=(n,),
        in_specs=[
            pl.BlockSpec((1, 232, 384), lambda i: (i, 0, 0)),
            pl.BlockSpec((9, 384, 256), lambda i: (0, 0, 0)),
            pl.BlockSpec((1, 256), lambda i: (0, 0)),
        ],
        out_specs=pl.BlockSpec((1, 232, 256), lambda i: (i, 0, 0)),
        compiler_params=par,
    )(h, w4, c4b.reshape(1, 256).astype(F32))

    h = pl.pallas_call(
        _conv5_pool3_kernel,
        out_shape=jax.ShapeDtypeStruct((n, 36, 256), BF),
        grid=(n,),
        in_specs=[
            pl.BlockSpec((1, 232, 256), lambda i: (i, 0, 0)),
            pl.BlockSpec((9, 256, 256), lambda i: (0, 0, 0)),
            pl.BlockSpec((1, 256), lambda i: (0, 0)),
        ],
        out_specs=pl.BlockSpec((1, 36, 256), lambda i: (i, 0, 0)),
        scratch_shapes=[pltpu.VMEM((200, 128), F32), pltpu.VMEM((200, 128), F32)],
        compiler_params=par,
    )(h, w5, c5b.reshape(1, 256).astype(F32))

    flat = h.transpose(0, 2, 1).reshape(n, 9216)

    h1 = _fc(flat, fc1w, fc1b, relu=True, tn=512, out_dtype=BF)
    h2 = _fc(h1, fc2w, fc2b, relu=True, tn=512, out_dtype=BF)
    w3f = jnp.pad(fc3w, ((0, 0), (0, 114)))
    b3f = jnp.pad(fc3b, (0, 114))
    h3 = _fc(h2, w3f, b3f, relu=False, tn=128, out_dtype=F32)
    return h3[:, :14]


# speedup vs baseline: 14.1437x; 1.0383x over previous
import functools

import jax
import jax.numpy as jnp
from jax.experimental import pallas as pl
from jax.experimental.pallas import tpu as pltpu

BF = jnp.bfloat16
F32 = jnp.float32


def _conv1_pool1_kernel(x_ref, w_ref, b_ref, o_ref, s_ref):
    o_ref[...] = jnp.zeros((1, 968, 128), BF)
    for band in range(3):
        base = 18 * band * 57
        acc = jnp.zeros((1083, 64), F32)
        for t in range(9):
            off = (t // 3) * 57 + (t % 3)
            acc += jnp.dot(x_ref[0, base + off:base + off + 1083, :], w_ref[t],
                           preferred_element_type=F32)
        s_ref[0:1083, :] = jnp.maximum(acc + b_ref[...], 0.0)
        for li in range(9):
            gi = 9 * band + li
            row = None
            for di in range(3):
                for dj in range(3):
                    piece = s_ref[pl.ds((2 * li + di) * 57 + dj, 27, 2), :]
                    row = piece if row is None else jnp.maximum(row, piece)
            o_ref[0, 64 + gi * 31:64 + gi * 31 + 27, 0:64] = row.astype(BF)


def _conv2_pool2_kernel(x_ref, w_ref, b_ref, o_ref, s5_ref, sa_ref, sb_ref):
    o_ref[...] = jnp.zeros((1, 232, 256), BF)
    for band in range(2):
        base = band * 372
        for kh in range(5):
            s5_ref[0:472, kh * 128:kh * 128 + 128] = \
                x_ref[0, base + kh * 31:base + kh * 31 + 472, :]
        acc = jnp.zeros((465, 256), F32)
        for kw in range(5):
            acc += jnp.dot(s5_ref[kw:kw + 465, :], w_ref[kw],
                           preferred_element_type=F32)
        v = jnp.maximum(acc + b_ref[...], 0.0)
        sa_ref[0:465, :] = v[:, 0:128]
        sb_ref[0:465, :] = v[:, 128:256]
        for li in range(7):
            gi = 6 * band + li
            for half, s_ref in enumerate((sa_ref, sb_ref)):
                row = None
                for di in range(3):
                    for dj in range(3):
                        piece = s_ref[pl.ds((2 * li + di) * 31 + dj, 13, 2), :]
                        row = piece if row is None else jnp.maximum(row, piece)
                o_ref[0, 16 + gi * 15:16 + gi * 15 + 13,
                      half * 128:half * 128 + 128] = row.astype(BF)


def _conv3x3_kernel(x_ref, w_ref, b_ref, o_ref, *, cout):
    acc = jnp.zeros((195, cout), F32)
    for t in range(9):
        off = (t // 3) * 15 + (t % 3)
        acc += jnp.dot(x_ref[0, off:off + 195, :], w_ref[t],
                       preferred_element_type=F32)
    v = jnp.maximum(acc + b_ref[...], 0.0).astype(BF)
    col = jax.lax.broadcasted_iota(jnp.int32, (195, cout), 0) % 15
    v = jnp.where(col < 13, v, jnp.zeros_like(v))
    o_ref[0, 0:16, :] = jnp.zeros((16, cout), BF)
    o_ref[0, 16:211, :] = v
    o_ref[0, 211:232, :] = jnp.zeros((21, cout), BF)


def _conv5_pool3_kernel(x_ref, w_ref, b_ref, o_ref, sa_ref, sb_ref):
    acc = jnp.zeros((195, 256), F32)
    for t in range(9):
        off = (t // 3) * 15 + (t % 3)
        acc += jnp.dot(x_ref[0, off:off + 195, :], w_ref[t],
                       preferred_element_type=F32)
    v = jnp.maximum(acc + b_ref[...], 0.0)
    sa_ref[0:195, :] = v[:, 0:128]
    sb_ref[0:195, :] = v[:, 128:256]
    for i in range(6):
        for half, s_ref in enumerate((sa_ref, sb_ref)):
            row = None
            for di in range(3):
                for dj in range(3):
                    piece = s_ref[pl.ds((2 * i + di) * 15 + dj, 6, 2), :]
                    row = piece if row is None else jnp.maximum(row, piece)
            o_ref[0, 6 * i:6 * i + 6, half * 128:half * 128 + 128] = row.astype(BF)


def _fc_kernel(a_ref, w_ref, b_ref, o_ref, *, relu):
    r = jnp.dot(a_ref[...], w_ref[...], preferred_element_type=F32) + b_ref[...]
    if relu:
        r = jnp.maximum(r, 0.0)
    o_ref[...] = r.astype(o_ref.dtype)


def _fc(a, w, b, *, relu, tn, out_dtype):
    M, K = a.shape
    N = w.shape[1]
    return pl.pallas_call(
        functools.partial(_fc_kernel, relu=relu),
        out_shape=jax.ShapeDtypeStruct((M, N), out_dtype),
        grid=(N // tn,),
        in_specs=[
            pl.BlockSpec((M, K), lambda j: (0, 0)),
            pl.BlockSpec((K, tn), lambda j: (0, j)),
            pl.BlockSpec((1, tn), lambda j: (0, j)),
        ],
        out_specs=pl.BlockSpec((M, tn), lambda j: (0, j)),
        compiler_params=pltpu.CompilerParams(
            dimension_semantics=("parallel",),
            vmem_limit_bytes=48 * 1024 * 1024),
    )(a, w, b.reshape(1, N).astype(F32))


def kernel(x, c1w, c1b, c2w, c2b, c3w, c3b, c4w, c4b, c5w, c5b,
           fc1w, fc1b, fc2w, fc2b, fc3w, fc3b):
    n = x.shape[0]

    xa = jnp.pad(x.transpose(0, 2, 3, 1).astype(BF),
                 ((0, 0), (2, 2), (2, 2), (0, 0)))
    phases = [xa[:, uh::4, :, :].reshape(n, 3249, 12) for uh in range(4)]
    xs = jnp.pad(jnp.concatenate(phases, axis=-1), ((0, 0), (0, 7), (0, 0)))

    w1 = jnp.pad(c1w, ((0, 1), (0, 1), (0, 0), (0, 0)))
    w1 = w1.reshape(3, 4, 3, 4, 3, 64).transpose(0, 2, 1, 3, 4, 5)
    w1 = w1.reshape(9, 48, 64)

    w2 = jnp.pad(c2w, ((0, 0), (0, 0), (0, 64), (0, 64)))
    w2 = w2.transpose(1, 0, 2, 3).reshape(5, 640, 256)
    b2 = jnp.pad(c2b, (0, 64))
    w3 = jnp.pad(c3w, ((0, 0), (0, 0), (0, 64), (0, 0))).reshape(9, 256, 384)
    w4 = c4w.reshape(9, 384, 256)
    w5 = c5w.reshape(9, 256, 256)

    par = pltpu.CompilerParams(dimension_semantics=("parallel",))

    h = pl.pallas_call(
        _conv1_pool1_kernel,
        out_shape=jax.ShapeDtypeStruct((n, 968, 128), BF),
        grid=(n,),
        in_specs=[
            pl.BlockSpec((1, 3256, 48), lambda i: (i, 0, 0)),
            pl.BlockSpec((9, 48, 64), lambda i: (0, 0, 0)),
            pl.BlockSpec((1, 64), lambda i: (0, 0)),
        ],
        out_specs=pl.BlockSpec((1, 968, 128), lambda i: (i, 0, 0)),
        scratch_shapes=[pltpu.VMEM((1088, 64), F32)],
        compiler_params=par,
    )(xs, w1, c1b.reshape(1, 64).astype(F32))

    h = pl.pallas_call(
        _conv2_pool2_kernel,
        out_shape=jax.ShapeDtypeStruct((n, 232, 256), BF),
        grid=(n,),
        in_specs=[
            pl.BlockSpec((1, 968, 128), lambda i: (i, 0, 0)),
            pl.BlockSpec((5, 640, 256), lambda i: (0, 0, 0)),
            pl.BlockSpec((1, 256), lambda i: (0, 0)),
        ],
        out_specs=pl.BlockSpec((1, 232, 256), lambda i: (i, 0, 0)),
        scratch_shapes=[pltpu.VMEM((472, 640), BF),
                        pltpu.VMEM((472, 128), F32), pltpu.VMEM((472, 128), F32)],
        compiler_params=par,
    )(h, w2, b2.reshape(1, 256).astype(F32))

    h = pl.pallas_call(
        functools.partial(_conv3x3_kernel, cout=384),
        out_shape=jax.ShapeDtypeStruct((n, 232, 384), BF),
        grid=(n,),
        in_specs=[
            pl.BlockSpec((1, 232, 256), lambda i: (i, 0, 0)),
            pl.BlockSpec((9, 256, 384), lambda i: (0, 0, 0)),
            pl.BlockSpec((1, 384), lambda i: (0, 0)),
        ],
        out_specs=pl.BlockSpec((1, 232, 384), lambda i: (i, 0, 0)),
        compiler_params=par,
    )(h, w3, c3b.reshape(1, 384).astype(F32))

    h = pl.pallas_call(
        functools.partial(_conv3x3_kernel, cout=256),
        out_shape=jax.ShapeDtypeStruct((n, 232, 256), BF),
        grid=(n,),
        in_specs=[
            pl.BlockSpec((1, 232, 384), lambda i: (i, 0, 0)),
            pl.BlockSpec((9, 384, 256), lambda i: (0, 0, 0)),
            pl.BlockSpec((1, 256), lambda i: (0, 0)),
        ],
        out_specs=pl.BlockSpec((1, 232, 256), lambda i: (i, 0, 0)),
        compiler_params=par,
    )(h, w4, c4b.reshape(1, 256).astype(F32))

    h = pl.pallas_call(
        _conv5_pool3_kernel,
        out_shape=jax.ShapeDtypeStruct((n, 36, 256), BF),
        grid=(n,),
        in_specs=[
            pl.BlockSpec((1, 232, 256), lambda i: (i, 0, 0)),
            pl.BlockSpec((9, 256, 256), lambda i: (0, 0, 0)),
            pl.BlockSpec((1, 256), lambda i: (0, 0)),
        ],
        out_specs=pl.BlockSpec((1, 36, 256), lambda i: (i, 0, 0)),
        scratch_shapes=[pltpu.VMEM((200, 128), F32), pltpu.VMEM((200, 128), F32)],
        compiler_params=par,
    )(h, w5, c5b.reshape(1, 256).astype(F32))

    flat = h.transpose(0, 2, 1).reshape(n, 9216)

    h1 = _fc(flat, fc1w, fc1b, relu=True, tn=512, out_dtype=BF)
    h2 = _fc(h1, fc2w, fc2b, relu=True, tn=512, out_dtype=BF)
    w3f = jnp.pad(fc3w, ((0, 0), (0, 114)))
    b3f = jnp.pad(fc3b, (0, 114))
    h3 = _fc(h2, w3f, b3f, relu=False, tn=128, out_dtype=F32)
    return h3[:, :14]
